# baseline (device time: 8930 ns/iter reference)
import jax
import jax.numpy as jnp
from jax import lax
from jax.experimental import pallas as pl
from jax.experimental.pallas import tpu as pltpu

N_CHUNKS = 4


def kernel(x, dy, gamma):
    m_per, d = x.shape
    mc = m_per // N_CHUNKS

    def body(x_hbm, dy_hbm, gamma_hbm, out_ref,
             xb, dyb, comm_ref, load_sems, out_sem, send_sem, recv_sem):
        my_x = lax.axis_index("x")
        my_y = lax.axis_index("y")
        my_z = lax.axis_index("z")
        peer = (my_x, 1 - my_y, my_z)

        barrier_sem = pltpu.get_barrier_semaphore()
        pl.semaphore_signal(
            barrier_sem, inc=1, device_id=peer,
            device_id_type=pl.DeviceIdType.MESH,
        )

        loads = []
        for k in range(N_CHUNKS):
            rows = pl.ds(k * mc, mc)
            cx = pltpu.make_async_copy(
                x_hbm.at[rows, :], xb.at[k], load_sems.at[2 * k])
            cdy = pltpu.make_async_copy(
                dy_hbm.at[rows, :], dyb.at[k], load_sems.at[2 * k + 1])
            cx.start()
            cdy.start()
            loads.append((cx, cdy))

        dg = jnp.zeros((1, d), jnp.float32)
        db = jnp.zeros((1, d), jnp.float32)
        for k in range(N_CHUNKS):
            cx, cdy = loads[k]
            cx.wait()
            cdy.wait()
            xv = xb[k]
            dyv = dyb[k]
            mu = jnp.mean(xv, axis=1, keepdims=True)
            var = jnp.mean((xv - mu) * (xv - mu), axis=1, keepdims=True)
            rstd = lax.rsqrt(var + 1e-5)
            xhat = ((xv - mu) * rstd).astype(jnp.bfloat16)
            dg = dg + jnp.sum(dyv.astype(jnp.bfloat16) * xhat, axis=0,
                              keepdims=True, dtype=jnp.float32)
            db = db + jnp.sum(dyv, axis=0, keepdims=True)
        comm_ref[0, :, :] = jnp.concatenate([dg, db], axis=0)

        pl.semaphore_wait(barrier_sem, 1)

        rdma = pltpu.make_async_remote_copy(
            src_ref=comm_ref.at[0],
            dst_ref=comm_ref.at[1],
            send_sem=send_sem,
            recv_sem=recv_sem,
            device_id=peer,
            device_id_type=pl.DeviceIdType.MESH,
        )
        rdma.start()
        rdma.wait()

        comm_ref[0, :, :] = comm_ref[0, :, :] + comm_ref[1, :, :]
        out_copy = pltpu.make_async_copy(comm_ref.at[0], out_ref, out_sem)
        out_copy.start()
        out_copy.wait()

    return pl.pallas_call(
        body,
        out_shape=jax.ShapeDtypeStruct((2, d), jnp.float32),
        in_specs=[
            pl.BlockSpec(memory_space=pl.ANY),
            pl.BlockSpec(memory_space=pl.ANY),
            pl.BlockSpec(memory_space=pl.ANY),
        ],
        out_specs=pl.BlockSpec(memory_space=pl.ANY),
        scratch_shapes=[
            pltpu.VMEM((N_CHUNKS, mc, d), jnp.float32),
            pltpu.VMEM((N_CHUNKS, mc, d), jnp.float32),
            pltpu.VMEM((2, 2, d), jnp.float32),
            pltpu.SemaphoreType.DMA((2 * N_CHUNKS,)),
            pltpu.SemaphoreType.DMA,
            pltpu.SemaphoreType.DMA,
            pltpu.SemaphoreType.DMA,
        ],
        compiler_params=pltpu.CompilerParams(collective_id=0),
    )(x, dy, gamma)


# device time: 6710 ns/iter; 1.3308x vs baseline; 1.3308x over previous
import jax
import jax.numpy as jnp
from jax import lax
from jax.experimental import pallas as pl
from jax.experimental.pallas import tpu as pltpu

N_CHUNKS = 4


def kernel(x, dy, gamma):
    m_per, d = x.shape
    mc = m_per // N_CHUNKS

    def body(x_hbm, dy_hbm, gamma_hbm, out_ref,
             xb, dyb, comm_ref, load_sems, out_sem, send_sem, recv_sem):
        my_x = lax.axis_index("x")
        my_y = lax.axis_index("y")
        my_z = lax.axis_index("z")
        peer = (my_x, 1 - my_y, my_z)

        barrier_sem = pltpu.get_barrier_semaphore()
        pl.semaphore_signal(
            barrier_sem, inc=1, device_id=peer,
            device_id_type=pl.DeviceIdType.MESH,
        )

        loads = []
        for k in range(N_CHUNKS):
            rows = pl.ds(k * mc, mc)
            cx = pltpu.make_async_copy(
                x_hbm.at[rows, :], xb.at[k], load_sems.at[2 * k])
            cdy = pltpu.make_async_copy(
                dy_hbm.at[rows, :], dyb.at[k], load_sems.at[2 * k + 1])
            cx.start()
            cdy.start()
            loads.append((cx, cdy))

        dg = jnp.zeros((1, d), jnp.float32)
        db = jnp.zeros((1, d), jnp.float32)
        for k in range(N_CHUNKS):
            cx, cdy = loads[k]
            cx.wait()
            cdy.wait()
            xv = xb[k]
            dyv = dyb[k]
            mu = jnp.mean(xv, axis=1, keepdims=True)
            var = jnp.mean((xv - mu) * (xv - mu), axis=1, keepdims=True)
            rstd = lax.rsqrt(var + 1e-5)
            xhat = ((xv - mu) * rstd).astype(jnp.bfloat16)
            dg = dg + jnp.sum(dyv.astype(jnp.bfloat16) * xhat, axis=0,
                              keepdims=True, dtype=jnp.float32)
            db = db + jnp.sum(dyv, axis=0, keepdims=True)
        comm_ref[0, :, :] = jnp.concatenate([dg, db], axis=0)

        pl.semaphore_wait(barrier_sem, 1)

        rdma = pltpu.make_async_remote_copy(
            src_ref=comm_ref.at[0],
            dst_ref=comm_ref.at[1],
            send_sem=send_sem,
            recv_sem=recv_sem,
            device_id=peer,
            device_id_type=pl.DeviceIdType.MESH,
        )
        rdma.start()
        rdma.wait()

        comm_ref[0, :, :] = comm_ref[0, :, :] + comm_ref[1, :, :]
        out_copy = pltpu.make_async_copy(comm_ref.at[0], out_ref, out_sem)
        out_copy.start()
        out_copy.wait()

    x = pltpu.with_memory_space_constraint(x, pltpu.MemorySpace.HBM)
    dy = pltpu.with_memory_space_constraint(dy, pltpu.MemorySpace.HBM)
    gamma = pltpu.with_memory_space_constraint(gamma, pltpu.MemorySpace.HBM)

    return pl.pallas_call(
        body,
        out_shape=jax.ShapeDtypeStruct((2, d), jnp.float32),
        in_specs=[
            pl.BlockSpec(memory_space=pltpu.MemorySpace.HBM),
            pl.BlockSpec(memory_space=pltpu.MemorySpace.HBM),
            pl.BlockSpec(memory_space=pltpu.MemorySpace.HBM),
        ],
        out_specs=pl.BlockSpec(memory_space=pltpu.MemorySpace.HBM),
        scratch_shapes=[
            pltpu.VMEM((N_CHUNKS, mc, d), jnp.float32),
            pltpu.VMEM((N_CHUNKS, mc, d), jnp.float32),
            pltpu.VMEM((2, 2, d), jnp.float32),
            pltpu.SemaphoreType.DMA((2 * N_CHUNKS,)),
            pltpu.SemaphoreType.DMA,
            pltpu.SemaphoreType.DMA,
            pltpu.SemaphoreType.DMA,
        ],
        compiler_params=pltpu.CompilerParams(collective_id=0),
    )(x, dy, gamma)
